# baseline (speedup 1.0000x reference)
import jax
import jax.numpy as jnp
from jax.experimental import pallas as pl
from jax.experimental.pallas import tpu as pltpu

_LAYER_CFGS = [
    (8, 1, -1, 0, 48, 24, 4, False),
    (12, 2, 384, 48, 96, 12, 2, False),
    (16, 2, 128, 96, 192, 24, 2, False),
    (16, 3, -1, 192, 384, 48, 2, True),
]


def _elu(x):
    return jnp.where(x > 0, x, jnp.exp(x) - 1.0)


def _xconv_block(pts, repT, catT, w, K, D, dm, with_global):
    N = pts.shape[0]
    P = repT.shape[1]
    Ccat = catT.shape[0]
    Cin = Ccat - 3
    KD = K * D

    def _dist2():
        dx = pts[:, 0:1] - repT[0:1, :]
        dy = pts[:, 1:2] - repT[1:2, :]
        dz = pts[:, 2:3] - repT[2:3, :]
        return (dx * dx + dy * dy) + dz * dz

    inf = jnp.float32(jnp.inf)
    ones_cnt = jnp.ones((1, N), jnp.float32)
    d2m = _dist2()
    nn_fast = []
    cntmax = jnp.zeros((1, P), jnp.float32)
    for t in range(KD):
        m = jnp.min(d2m, axis=0, keepdims=True)
        eq = d2m == m
        eqf = eq.astype(jnp.float32)
        if t % D == 0:
            nn_fast.append(jnp.dot(catT, eqf,
                                   preferred_element_type=jnp.float32))
        cnt = jnp.dot(ones_cnt, eqf, preferred_element_type=jnp.float32)
        cntmax = jnp.maximum(cntmax, cnt)
        d2m = jnp.where(eq, inf, d2m)
    fast = jnp.concatenate(nn_fast, axis=0)
    maxcnt = jnp.max(cntmax)

    def _exact(_):
        iota = jax.lax.broadcasted_iota(jnp.int32, (N, P), 0)
        d2e = _dist2()
        outs = []
        for t in range(KD):
            m = jnp.min(d2e, axis=0, keepdims=True)
            cand = jnp.where(d2e == m, iota, N)
            sel = jnp.min(cand, axis=0, keepdims=True)
            onehot = iota == sel
            if t % D == 0:
                outs.append(jnp.dot(catT, onehot.astype(jnp.float32),
                                    preferred_element_type=jnp.float32))
            d2e = jnp.where(onehot, inf, d2e)
        return jnp.concatenate(outs, axis=0)

    gathered = jax.lax.cond(maxcnt > 1.5, _exact, lambda _: fast, None)
    nnT = [gathered[k * Ccat:(k + 1) * Ccat] for k in range(K)]

    locT = [g[0:3, :] - repT for g in nnT]
    lfT = jnp.concatenate(locT, axis=0)
    locT_all = jnp.concatenate(locT, axis=1)

    Wd1 = w['Wd1']
    h = (Wd1[:, 0:1] * locT_all[0:1, :] + Wd1[:, 1:2] * locT_all[1:2, :]
         + Wd1[:, 2:3] * locT_all[2:3, :]) + w['bd1']
    h = _elu(h)
    h = _elu(jnp.dot(w['Wd2'], h, preferred_element_type=jnp.float32)
             + w['bd2'])

    X = _elu(jnp.dot(w['Wx0'], lfT, preferred_element_type=jnp.float32)
             + w['bx0'])
    X = _elu(jnp.dot(w['Wx1'], X, preferred_element_type=jnp.float32)
             + w['bx1'])
    X = jnp.dot(w['Wx2'], X, preferred_element_type=jnp.float32) + w['bx2']

    H = []
    for j in range(K):
        hj = h[:, j * P:(j + 1) * P]
        if Cin:
            hj = jnp.concatenate([hj, nnT[j][3:, :]], axis=0)
        H.append(hj)

    fX = []
    for k in range(K):
        acc = X[k * K:k * K + 1, :] * H[0]
        for j in range(1, K):
            acc = acc + X[k * K + j:k * K + j + 1, :] * H[j]
        fX.append(acc)

    Wdw = w['Wdw']
    dws = []
    for mi in range(dm):
        wk = Wdw[mi]
        acc = fX[0] * wk[:, 0:1]
        for k in range(1, K):
            acc = acc + fX[k] * wk[:, k:k + 1]
        dws.append(acc)
    dwT = jnp.concatenate(dws, axis=0)
    out = jnp.dot(w['Wpw'], dwT, preferred_element_type=jnp.float32) \
        + w['bpw']

    if with_global:
        Wg1 = w['Wg1']
        g = (Wg1[:, 0:1] * repT[0:1, :] + Wg1[:, 1:2] * repT[1:2, :]
             + Wg1[:, 2:3] * repT[2:3, :]) + w['bg1']
        g = _elu(g)
        g = _elu(jnp.dot(w['Wg2'], g, preferred_element_type=jnp.float32)
                 + w['bg2'])
        out = jnp.concatenate([g, out], axis=0)

    return out


_WKEYS = ['Wd1', 'bd1', 'Wd2', 'bd2', 'Wx0', 'bx0', 'Wx1', 'bx1',
          'Wx2', 'bx2', 'Wdw', 'Wpw', 'bpw']
_GKEYS = ['Wg1', 'bg1', 'Wg2', 'bg2']


def _prep_weights(p, dm, with_global):
    Cmid = p['Wdw'].shape[1]
    Cout = p['Wpw'].shape[1]

    def bcol(b):
        return b.reshape(-1, 1)

    w = {
        'Wd1': p['Wd1'].T, 'bd1': bcol(p['bd1']),
        'Wd2': p['Wd2'].T, 'bd2': bcol(p['bd2']),
        'Wx0': p['Wx0'].T, 'bx0': bcol(p['bx0']),
        'Wx1': p['Wx1'].T, 'bx1': bcol(p['bx1']),
        'Wx2': p['Wx2'].T, 'bx2': bcol(p['bx2']),
        'Wdw': jnp.transpose(p['Wdw'], (2, 1, 0)),
        'Wpw': p['Wpw'].reshape(Cmid, dm, Cout).transpose(1, 0, 2)
               .reshape(dm * Cmid, Cout).T,
        'bpw': bcol(p['bpw']),
    }
    if with_global:
        w.update({'Wg1': p['Wg1'].T, 'bg1': bcol(p['bg1']),
                  'Wg2': p['Wg2'].T, 'bg2': bcol(p['bg2'])})
    return w


def kernel(pc, params):
    B, N0, _ = pc.shape
    pcT = jnp.transpose(pc, (0, 2, 1))

    flat_ws = []
    for (K, D, P, _Ci, _Co, _Cd, dm, wg), p in zip(_LAYER_CFGS, params):
        w = _prep_weights(p, dm, wg)
        for k in _WKEYS + (_GKEYS if wg else []):
            flat_ws.append(w[k])

    Cfin = 480

    def body(pts_ref, pcT_ref, *refs):
        out_ref = refs[-1]
        wrefs = refs[:-1]
        pts0 = pts_ref[0]
        pcT0 = pcT_ref[0]

        idx = 0
        ftsT = None
        N = N0
        for (K, D, P, _Ci, _Co, _Cd, dm, wg) in _LAYER_CFGS:
            nkeys = len(_WKEYS) + (len(_GKEYS) if wg else 0)
            keys = _WKEYS + (_GKEYS if wg else [])
            w = {k: wrefs[idx + i][...] for i, k in enumerate(keys)}
            idx += nkeys
            P_l = N if P == -1 else P
            pcT_l = pcT0[:, :N]
            catT = pcT_l if ftsT is None else \
                jnp.concatenate([pcT_l, ftsT], axis=0)
            ftsT = _xconv_block(pts0[:N, :], pcT0[:, :P_l], catT,
                                w, K, D, dm, wg)
            N = P_l

        out_ref[0] = ftsT

    in_specs = [
        pl.BlockSpec((1, N0, 3), lambda b: (b, 0, 0)),
        pl.BlockSpec((1, 3, N0), lambda b: (b, 0, 0)),
    ] + [pl.BlockSpec(a.shape, lambda b, _z=(0,) * a.ndim: _z)
         for a in flat_ws]

    outT = pl.pallas_call(
        body,
        grid=(B,),
        in_specs=in_specs,
        out_specs=pl.BlockSpec((1, Cfin, 128), lambda b: (b, 0, 0)),
        out_shape=jax.ShapeDtypeStruct((B, Cfin, 128), jnp.float32),
        compiler_params=pltpu.CompilerParams(
            dimension_semantics=("parallel",)),
    )(pc, pcT, *flat_ws)

    return jnp.transpose(outT, (0, 2, 1))


# speedup vs baseline: 1.7672x; 1.7672x over previous
import jax
import jax.numpy as jnp
from jax.experimental import pallas as pl
from jax.experimental.pallas import tpu as pltpu

_LAYER_CFGS = [
    (8, 1, -1, 0, 48, 24, 4, False),
    (12, 2, 384, 48, 96, 12, 2, False),
    (16, 2, 128, 96, 192, 24, 2, False),
    (16, 3, -1, 192, 384, 48, 2, True),
]


def _elu(x):
    return jnp.where(x > 0, x, jnp.exp(x) - 1.0)


def _make_layer_kernel(K, D, dm, Cin, with_global, N, P_tile):
    KD = K * D

    def body(pts_ref, repT_ref, catT_ref, Wd1_ref, bd1_ref, Wd2_ref, bd2_ref,
             Wx0_ref, bx0_ref, Wx1_ref, bx1_ref, Wx2_ref, bx2_ref,
             Wdw_ref, Wpw_ref, bpw_ref, *rest):
        if with_global:
            Wg1_ref, bg1_ref, Wg2_ref, bg2_ref, out_ref = rest
        else:
            (out_ref,) = rest

        pts = pts_ref[0]
        repT = repT_ref[0]
        catT = catT_ref[0]
        Ccat = catT.shape[0]

        def _dist2():
            dx = pts[:, 0:1] - repT[0:1, :]
            dy = pts[:, 1:2] - repT[1:2, :]
            dz = pts[:, 2:3] - repT[2:3, :]
            return (dx * dx + dy * dy) + dz * dz

        iota = jax.lax.broadcasted_iota(jnp.int32, (N, P_tile), 0)
        inf = jnp.float32(jnp.inf)
        ones_cnt = jnp.ones((1, N), jnp.float32)
        d2m = _dist2()
        nn_fast = []
        for t in range(KD):
            m = jnp.min(d2m, axis=0, keepdims=True)
            eq = d2m == m
            if t % D == 0:
                eqf = eq.astype(jnp.float32)
                gat = jnp.dot(catT, eqf, preferred_element_type=jnp.float32)
                nn_fast.append(gat)
            d2m = jnp.where(eq, inf, d2m)
        fast = jnp.concatenate(nn_fast, axis=0)
        removed = (d2m == inf).astype(jnp.float32)
        maxcnt = jnp.max(jnp.dot(ones_cnt, removed,
                                 preferred_element_type=jnp.float32))

        def _exact(_):
            d2e = _dist2()
            outs = []
            for t in range(KD):
                m = jnp.min(d2e, axis=0, keepdims=True)
                cand = jnp.where(d2e == m, iota, N)
                sel = jnp.min(cand, axis=0, keepdims=True)
                onehot = iota == sel
                if t % D == 0:
                    outs.append(jnp.dot(catT, onehot.astype(jnp.float32),
                                        preferred_element_type=jnp.float32))
                d2e = jnp.where(onehot, inf, d2e)
            return jnp.concatenate(outs, axis=0)

        gathered = jax.lax.cond(maxcnt > KD + 0.5, _exact,
                                lambda _: fast, None)
        nnT = [gathered[k * Ccat:(k + 1) * Ccat] for k in range(K)]

        locT = [g[0:3, :] - repT for g in nnT]
        lfT = jnp.concatenate(locT, axis=0)
        locT_all = jnp.concatenate(locT, axis=1)

        Wd1 = Wd1_ref[...]
        h = (Wd1[:, 0:1] * locT_all[0:1, :] + Wd1[:, 1:2] * locT_all[1:2, :]
             + Wd1[:, 2:3] * locT_all[2:3, :]) + bd1_ref[...]
        h = _elu(h)
        h = _elu(jnp.dot(Wd2_ref[...], h, preferred_element_type=jnp.float32)
                 + bd2_ref[...])

        X = _elu(jnp.dot(Wx0_ref[...], lfT, preferred_element_type=jnp.float32)
                 + bx0_ref[...])
        X = _elu(jnp.dot(Wx1_ref[...], X, preferred_element_type=jnp.float32)
                 + bx1_ref[...])
        X = jnp.dot(Wx2_ref[...], X, preferred_element_type=jnp.float32) \
            + bx2_ref[...]

        H = []
        for j in range(K):
            hj = h[:, j * P_tile:(j + 1) * P_tile]
            if Cin:
                hj = jnp.concatenate([hj, nnT[j][3:, :]], axis=0)
            H.append(hj)

        fX = []
        for k in range(K):
            acc = X[k * K:k * K + 1, :] * H[0]
            for j in range(1, K):
                acc = acc + X[k * K + j:k * K + j + 1, :] * H[j]
            fX.append(acc)

        Wdw = Wdw_ref[...]
        dws = []
        for mi in range(dm):
            w = Wdw[mi]
            acc = fX[0] * w[:, 0:1]
            for k in range(1, K):
                acc = acc + fX[k] * w[:, k:k + 1]
            dws.append(acc)
        dwT = jnp.concatenate(dws, axis=0)
        out = jnp.dot(Wpw_ref[...], dwT, preferred_element_type=jnp.float32) \
            + bpw_ref[...]

        if with_global:
            Wg1 = Wg1_ref[...]
            g = (Wg1[:, 0:1] * repT[0:1, :] + Wg1[:, 1:2] * repT[1:2, :]
                 + Wg1[:, 2:3] * repT[2:3, :]) + bg1_ref[...]
            g = _elu(g)
            g = _elu(jnp.dot(Wg2_ref[...], g,
                             preferred_element_type=jnp.float32)
                     + bg2_ref[...])
            out = jnp.concatenate([g, out], axis=0)

        out_ref[0] = out

    return body


def _xconv_layer(pts, ptsT, ftsT, p, K, D, P, dm, with_global):
    B, N, _ = pts.shape
    Cin = 0 if ftsT is None else ftsT.shape[1]
    Cd = p['Wd1'].shape[1]
    Cmid = Cd + Cin
    Cout = p['Wpw'].shape[1]
    Cg = p['Wg1'].shape[1] if with_global else 0

    P_tile = 1024 if P > 384 else P

    catT = ptsT if ftsT is None else jnp.concatenate([ptsT, ftsT], axis=1)
    Wdw_p = jnp.transpose(p['Wdw'], (2, 1, 0))
    WpwT = p['Wpw'].reshape(Cmid, dm, Cout).transpose(1, 0, 2) \
        .reshape(dm * Cmid, Cout).T

    def bcol(b):
        return b.reshape(-1, 1)

    ops = [pts, ptsT, catT,
           p['Wd1'].T, bcol(p['bd1']), p['Wd2'].T, bcol(p['bd2']),
           p['Wx0'].T, bcol(p['bx0']), p['Wx1'].T, bcol(p['bx1']),
           p['Wx2'].T, bcol(p['bx2']),
           Wdw_p, WpwT, bcol(p['bpw'])]
    if with_global:
        ops += [p['Wg1'].T, bcol(p['bg1']), p['Wg2'].T, bcol(p['bg2'])]

    def wspec(arr):
        zeros = (0,) * arr.ndim
        return pl.BlockSpec(arr.shape, lambda b, j, _z=zeros: _z)

    in_specs = [
        pl.BlockSpec((1, N, 3), lambda b, j: (b, 0, 0)),
        pl.BlockSpec((1, 3, P_tile), lambda b, j: (b, 0, j)),
        pl.BlockSpec((1, 3 + Cin, N), lambda b, j: (b, 0, 0)),
    ] + [wspec(a) for a in ops[3:]]

    fn = _make_layer_kernel(K, D, dm, Cin, with_global, N, P_tile)
    return pl.pallas_call(
        fn,
        grid=(B, P // P_tile),
        in_specs=in_specs,
        out_specs=pl.BlockSpec((1, Cg + Cout, P_tile), lambda b, j: (b, 0, j)),
        out_shape=jax.ShapeDtypeStruct((B, Cg + Cout, P), jnp.float32),
        compiler_params=pltpu.CompilerParams(
            dimension_semantics=("parallel", "parallel")),
    )(*ops)


def kernel(pc, params):
    pcT = jnp.transpose(pc, (0, 2, 1))
    N = pc.shape[1]
    ftsT = None
    for (K, D, P, _Cin, _Cout, _Cd, dm, wg), p in zip(_LAYER_CFGS, params):
        P_l = N if P == -1 else P
        ftsT = _xconv_layer(pc[:, :N, :], pcT[:, :, :N], ftsT, p,
                            K, D, P_l, dm, wg)
        N = P_l
    return jnp.transpose(ftsT, (0, 2, 1))
